# baseline (speedup 1.0000x reference)
import functools

import jax
import jax.numpy as jnp
from jax import lax
from jax.experimental import pallas as pl
from jax.experimental.pallas import tpu as pltpu
from jax.experimental.pallas import tpu_sc as plsc

NC = 2
NS = 16
CH = 128
G = 8


@functools.partial(jax.jit, static_argnums=(4, 5, 6))
def _run(feature, src2, dst2, zeros, N, D, n_chunks):
    nup = -(-(N + 1) // (8 * NS)) * (8 * NS)
    zrows = nup // NS
    n_groups = n_chunks // G

    mesh = plsc.VectorSubcoreMesh(core_axis_name="c", subcore_axis_name="s")

    @functools.partial(
        pl.kernel,
        out_type=jax.ShapeDtypeStruct((NC, nup, D), jnp.float32),
        mesh=mesh,
        scratch_types=[
            pltpu.VMEM_SHARED((nup, D), jnp.float32),
            pltpu.VMEM((G, CH), jnp.int32),
            pltpu.VMEM((G, CH), jnp.int32),
            pltpu.VMEM((CH, D), jnp.float32),
            pltpu.SemaphoreType.DMA,
        ],
    )
    def k(feat_hbm, src_hbm, dst_hbm, zeros_hbm, part_hbm, acc, src_g, dst_g,
          rows_v, sem):
        c = lax.axis_index("c")
        s = lax.axis_index("s")
        wid = s * NC + c
        pltpu.sync_copy(zeros_hbm, acc.at[pl.ds(s * zrows, zrows)])
        plsc.subcore_barrier()

        def group(g, carry):
            row0 = wid * n_chunks + g * G
            pltpu.sync_copy(src_hbm.at[pl.ds(row0, G)], src_g)
            pltpu.sync_copy(dst_hbm.at[pl.ds(row0, G)], dst_g)
            for j in range(G):
                pltpu.async_copy(feat_hbm.at[src_g.at[j]], rows_v,
                                 sem).wait()
                pltpu.sync_copy(rows_v, acc.at[dst_g.at[j]], add=True)
            return carry

        lax.fori_loop(0, n_groups, group, 0)
        plsc.subcore_barrier()
        pltpu.sync_copy(acc.at[pl.ds(s * zrows, zrows)],
                        part_hbm.at[c].at[pl.ds(s * zrows, zrows)])

    part = k(feature, src2, dst2, zeros)

    rb = 1000

    def add_body(p_ref, o_ref):
        o_ref[...] = p_ref[0] + p_ref[1]

    return pl.pallas_call(
        add_body,
        grid=(N // rb,),
        in_specs=[pl.BlockSpec((NC, rb, D), lambda i: (0, i, 0))],
        out_specs=pl.BlockSpec((rb, D), lambda i: (i, 0)),
        out_shape=jax.ShapeDtypeStruct((N, D), jnp.float32),
    )(part)


def kernel(feature, edge_index):
    N, D = feature.shape
    E = edge_index.shape[1]
    nw = NC * NS
    n_chunks = -(-(-(-E // nw)) // (G * CH)) * G
    EP = n_chunks * CH * nw
    pad = EP - E
    src = jnp.concatenate(
        [edge_index[0].astype(jnp.int32), jnp.zeros((pad,), jnp.int32)])
    dst = jnp.concatenate(
        [edge_index[1].astype(jnp.int32), jnp.full((pad,), N, jnp.int32)])
    src2 = src.reshape(EP // CH, CH)
    dst2 = dst.reshape(EP // CH, CH)
    nup = -(-(N + 1) // (8 * NS)) * (8 * NS)
    zeros = jnp.zeros((nup // NS, D), jnp.float32)
    return _run(feature, src2, dst2, zeros, N, D, n_chunks)


# speedup vs baseline: 1.0065x; 1.0065x over previous
import functools

import jax
import jax.numpy as jnp
from jax import lax
from jax.experimental import pallas as pl
from jax.experimental.pallas import tpu as pltpu
from jax.experimental.pallas import tpu_sc as plsc

NC = 2
NS = 16
CH = 128
G = 8


@functools.partial(jax.jit, static_argnums=(4, 5, 6))
def _run(feature, src2, dst2, zeros, N, D, n_chunks):
    nup = -(-(N + 1) // (8 * NS)) * (8 * NS)
    zrows = nup // NS
    n_groups = n_chunks // G

    mesh = plsc.VectorSubcoreMesh(core_axis_name="c", subcore_axis_name="s")

    @functools.partial(
        pl.kernel,
        out_type=jax.ShapeDtypeStruct((NC, nup, D), jnp.float32),
        mesh=mesh,
        scratch_types=[
            pltpu.VMEM_SHARED((nup, D), jnp.float32),
            pltpu.VMEM((n_chunks * CH,), jnp.int32),
            pltpu.VMEM((n_chunks * CH,), jnp.int32),
            pltpu.VMEM((CH,), jnp.int32),
            pltpu.VMEM((CH,), jnp.int32),
            pltpu.VMEM((CH, D), jnp.float32),
            pltpu.SemaphoreType.DMA,
        ],
    )
    def k(feat_hbm, src_hbm, dst_hbm, zeros_hbm, part_hbm, acc, src_all,
          dst_all, src_v, dst_v, rows_v, sem):
        c = lax.axis_index("c")
        s = lax.axis_index("s")
        wid = s * NC + c
        base = wid * n_chunks * CH
        pltpu.sync_copy(src_hbm.at[pl.ds(base, n_chunks * CH)], src_all)
        pltpu.sync_copy(dst_hbm.at[pl.ds(base, n_chunks * CH)], dst_all)
        pltpu.sync_copy(zeros_hbm, acc.at[pl.ds(s * zrows, zrows)])
        plsc.subcore_barrier()

        def step(j, carry):
            off = j * CH
            for l in range(CH // 16):
                src_v[pl.ds(l * 16, 16)] = src_all[pl.ds(off + l * 16, 16)]
                dst_v[pl.ds(l * 16, 16)] = dst_all[pl.ds(off + l * 16, 16)]
            pltpu.async_copy(feat_hbm.at[src_v], rows_v, sem).wait()
            pltpu.sync_copy(rows_v, acc.at[dst_v], add=True)
            return carry

        lax.fori_loop(0, n_chunks, step, 0)
        plsc.subcore_barrier()
        pltpu.sync_copy(acc.at[pl.ds(s * zrows, zrows)],
                        part_hbm.at[c].at[pl.ds(s * zrows, zrows)])

    part = k(feature, src2, dst2, zeros)

    rb = 1000

    def add_body(p_ref, o_ref):
        o_ref[...] = p_ref[0] + p_ref[1]

    return pl.pallas_call(
        add_body,
        grid=(N // rb,),
        in_specs=[pl.BlockSpec((NC, rb, D), lambda i: (0, i, 0))],
        out_specs=pl.BlockSpec((rb, D), lambda i: (i, 0)),
        out_shape=jax.ShapeDtypeStruct((N, D), jnp.float32),
    )(part)


def kernel(feature, edge_index):
    N, D = feature.shape
    E = edge_index.shape[1]
    nw = NC * NS
    n_chunks = -(-(-(-E // nw)) // (G * CH)) * G
    EP = n_chunks * CH * nw
    pad = EP - E
    src = jnp.concatenate(
        [edge_index[0].astype(jnp.int32), jnp.zeros((pad,), jnp.int32)])
    dst = jnp.concatenate(
        [edge_index[1].astype(jnp.int32), jnp.full((pad,), N, jnp.int32)])
    nup = -(-(N + 1) // (8 * NS)) * (8 * NS)
    zeros = jnp.zeros((nup // NS, D), jnp.float32)
    return _run(feature, src, dst, zeros, N, D, n_chunks)
